# baseline (speedup 1.0000x reference)
import jax
import jax.numpy as jnp
from jax import lax
from jax.experimental import pallas as pl
from jax.experimental.pallas import tpu as pltpu
from jax.experimental.pallas import tpu_sc as plsc

_B, _S, _P = 16, 4096, 8192
_NC, _NS, _L = 2, 16, 16
_NW = _NC * _NS
_PPW = _B * _P // _NW
_SB = _S // 128
_PB = _PPW // 128
_UNROLL = 8
_GROUPS = _PPW // _L
_ITERS = _GROUPS // _UNROLL
_SCALE = 1.0 / (_B * _S)


def _dtw_body(preds_hbm, targets_hbm, paths_hbm, out_hbm,
              preds_v, targs_v, path_v, acc_v, shared_v, sem_p, sem_t, sem_i):
    sid = lax.axis_index("s")
    b = sid
    half = lax.axis_index("c")

    @pl.when(sid == 0)
    def _init():
        acc_v[...] = jnp.zeros((_L,), jnp.float32)
        pltpu.sync_copy(acc_v, shared_v)

    plsc.subcore_barrier()

    cp_p = pltpu.make_async_copy(preds_hbm.at[b], preds_v, sem_p)
    cp_t = pltpu.make_async_copy(targets_hbm.at[b], targs_v, sem_t)
    cp_i = pltpu.make_async_copy(
        paths_hbm.at[b, pl.ds(half * 2 * _PB, 2 * _PB)], path_v, sem_i)
    cp_p.start()
    cp_t.start()
    cp_i.start()
    cp_p.wait()
    cp_t.wait()
    cp_i.wait()

    def step(k, acc):
        gbase = k * _UNROLL
        for u in range(_UNROLL):
            g = gbase + u
            blk = g // 8
            w = (g % 8) * _L
            iv = path_v[2 * blk, pl.ds(w, _L)]
            jv = path_v[2 * blk + 1, pl.ds(w, _L)]
            ri = (iv >> 7) << 1
            ci = iv & 127
            rj = (jv >> 7) << 1
            cj = jv & 127
            px = plsc.load_gather(preds_v, [ri, ci])
            py = plsc.load_gather(preds_v, [ri + 1, ci])
            tx = plsc.load_gather(targs_v, [rj, cj])
            ty = plsc.load_gather(targs_v, [rj + 1, cj])
            acc = acc + (jnp.abs(px - tx) + jnp.abs(py - ty))
        return acc

    acc = lax.fori_loop(0, _ITERS, step, jnp.zeros((_L,), jnp.float32))
    acc_v[...] = acc * _SCALE
    pltpu.sync_copy(acc_v, shared_v.at[lax.iota(jnp.int32, _L)], add=True)
    plsc.subcore_barrier()

    @pl.when(sid == 0)
    def _emit():
        pltpu.sync_copy(shared_v, acc_v)
        pltpu.sync_copy(acc_v, out_hbm.at[half])


def kernel(preds, targets, paths):
    pv = preds.reshape(_B, _SB, 128, 2).transpose(0, 1, 3, 2)
    pv = pv.reshape(_B, 2 * _SB, 128)
    tv = targets.reshape(_B, _SB, 128, 2).transpose(0, 1, 3, 2)
    tv = tv.reshape(_B, 2 * _SB, 128)
    av = paths.reshape(_B, _P // 128, 128, 2).transpose(0, 1, 3, 2)
    av = av.reshape(_B, 2 * (_P // 128), 128)
    partials = pl.kernel(
        _dtw_body,
        out_type=jax.ShapeDtypeStruct((_NC, _L), jnp.float32),
        mesh=plsc.VectorSubcoreMesh(core_axis_name="c", subcore_axis_name="s"),
        compiler_params=pltpu.CompilerParams(needs_layout_passes=False),
        scratch_types=[
            pltpu.VMEM((2 * _SB, 128), jnp.float32),
            pltpu.VMEM((2 * _SB, 128), jnp.float32),
            pltpu.VMEM((2 * _PB, 128), jnp.int32),
            pltpu.VMEM((_L,), jnp.float32),
            pltpu.VMEM_SHARED((_L,), jnp.float32),
            pltpu.SemaphoreType.DMA,
            pltpu.SemaphoreType.DMA,
            pltpu.SemaphoreType.DMA,
        ],
    )(pv, tv, av)
    return jnp.sum(partials)


# speedup vs baseline: 1.0025x; 1.0025x over previous
import jax
import jax.numpy as jnp
from jax import lax
from jax.experimental import pallas as pl
from jax.experimental.pallas import tpu as pltpu
from jax.experimental.pallas import tpu_sc as plsc

_B, _S, _P = 16, 4096, 8192
_NC, _NS, _L = 2, 16, 16
_NW = _NC * _NS
_PPW = _B * _P // _NW
_SB = _S // 128
_PB = _PPW // 128
_UNROLL = 8
_GROUPS = _PPW // _L
_ITERS = _GROUPS // _UNROLL
_SCALE = 1.0 / (_B * _S)


def _dtw_body(preds_hbm, targets_hbm, paths_hbm, out_hbm,
              preds_v, targs_v, path_v, acc_v, sem_p, sem_t, sem_i):
    sid = lax.axis_index("s")
    b = sid
    half = lax.axis_index("c")
    wid = sid * _NC + half

    cp_p = pltpu.make_async_copy(preds_hbm.at[b], preds_v, sem_p)
    cp_t = pltpu.make_async_copy(targets_hbm.at[b], targs_v, sem_t)
    cp_i = pltpu.make_async_copy(
        paths_hbm.at[b, pl.ds(half * 2 * _PB, 2 * _PB)], path_v, sem_i)
    cp_p.start()
    cp_t.start()
    cp_i.start()
    cp_p.wait()
    cp_t.wait()
    cp_i.wait()

    def step(g, acc):
        blk = g // 8
        w = (g % 8) * _L
        iv = path_v[2 * blk, pl.ds(w, _L)]
        jv = path_v[2 * blk + 1, pl.ds(w, _L)]
        ri = (iv >> 7) << 1
        ci = iv & 127
        rj = (jv >> 7) << 1
        cj = jv & 127
        px = plsc.load_gather(preds_v, [ri, ci])
        py = plsc.load_gather(preds_v, [ri + 1, ci])
        tx = plsc.load_gather(targs_v, [rj, cj])
        ty = plsc.load_gather(targs_v, [rj + 1, cj])
        return acc + (jnp.abs(px - tx) + jnp.abs(py - ty))

    acc = plsc.parallel_loop(
        0, _GROUPS, unroll=_UNROLL, carry=jnp.zeros((_L,), jnp.float32)
    )(step)
    acc_v[...] = acc * _SCALE
    pltpu.sync_copy(acc_v, out_hbm.at[wid])


def kernel(preds, targets, paths):
    pv = preds.reshape(_B, _SB, 128, 2).transpose(0, 1, 3, 2)
    pv = pv.reshape(_B, 2 * _SB, 128)
    tv = targets.reshape(_B, _SB, 128, 2).transpose(0, 1, 3, 2)
    tv = tv.reshape(_B, 2 * _SB, 128)
    av = paths.reshape(_B, _P // 128, 128, 2).transpose(0, 1, 3, 2)
    av = av.reshape(_B, 2 * (_P // 128), 128)
    partials = pl.kernel(
        _dtw_body,
        out_type=jax.ShapeDtypeStruct((_NW, _L), jnp.float32),
        mesh=plsc.VectorSubcoreMesh(core_axis_name="c", subcore_axis_name="s"),
        compiler_params=pltpu.CompilerParams(needs_layout_passes=False),
        scratch_types=[
            pltpu.VMEM((2 * _SB, 128), jnp.float32),
            pltpu.VMEM((2 * _SB, 128), jnp.float32),
            pltpu.VMEM((2 * _PB, 128), jnp.int32),
            pltpu.VMEM((_L,), jnp.float32),
            pltpu.SemaphoreType.DMA,
            pltpu.SemaphoreType.DMA,
            pltpu.SemaphoreType.DMA,
        ],
    )(pv, tv, av)
    return jnp.sum(partials)


# speedup vs baseline: 1.0069x; 1.0044x over previous
import jax
import jax.numpy as jnp
from jax import lax
from jax.experimental import pallas as pl
from jax.experimental.pallas import tpu as pltpu
from jax.experimental.pallas import tpu_sc as plsc

_B, _S, _P = 16, 4096, 8192
_NC, _NS, _L = 2, 16, 16
_NW = _NC * _NS
_PPW = _B * _P // _NW
_SB = _S // 128
_PB = _PPW // 128
_UNROLL = 8
_GROUPS = _PPW // _L
_ITERS = _GROUPS // _UNROLL
_SCALE = 1.0 / (_B * _S)


def _dtw_body(preds_hbm, targets_hbm, paths_hbm, out_hbm,
              preds_v, targs_v, path_v, acc_v, sem_p, sem_t, sem_i):
    sid = lax.axis_index("s")
    b = sid
    half = lax.axis_index("c")
    wid = sid * _NC + half

    cp_p = pltpu.make_async_copy(preds_hbm.at[b], preds_v, sem_p)
    cp_t = pltpu.make_async_copy(targets_hbm.at[b], targs_v, sem_t)
    cp_i = pltpu.make_async_copy(
        paths_hbm.at[b, pl.ds(half * 2 * _PB, 2 * _PB)], path_v, sem_i)
    cp_p.start()
    cp_t.start()
    cp_i.start()
    cp_p.wait()
    cp_t.wait()
    cp_i.wait()

    def step(k, acc):
        gbase = k * _UNROLL
        for u in range(_UNROLL):
            g = gbase + u
            blk = g // 8
            w = (g % 8) * _L
            iv = path_v[2 * blk, pl.ds(w, _L)]
            jv = path_v[2 * blk + 1, pl.ds(w, _L)]
            ri = (iv >> 7) << 1
            ci = iv & 127
            rj = (jv >> 7) << 1
            cj = jv & 127
            px = plsc.load_gather(preds_v, [ri, ci])
            py = plsc.load_gather(preds_v, [ri + 1, ci])
            tx = plsc.load_gather(targs_v, [rj, cj])
            ty = plsc.load_gather(targs_v, [rj + 1, cj])
            acc = acc + (jnp.abs(px - tx) + jnp.abs(py - ty))
        return acc

    acc = lax.fori_loop(0, _ITERS, step, jnp.zeros((_L,), jnp.float32))
    acc_v[...] = acc * _SCALE
    pltpu.sync_copy(acc_v, out_hbm.at[wid])


def kernel(preds, targets, paths):
    pv = preds.reshape(_B, _SB, 128, 2).transpose(0, 1, 3, 2)
    pv = pv.reshape(_B, 2 * _SB, 128)
    tv = targets.reshape(_B, _SB, 128, 2).transpose(0, 1, 3, 2)
    tv = tv.reshape(_B, 2 * _SB, 128)
    av = paths.reshape(_B, _P // 128, 128, 2).transpose(0, 1, 3, 2)
    av = av.reshape(_B, 2 * (_P // 128), 128)
    partials = pl.kernel(
        _dtw_body,
        out_type=jax.ShapeDtypeStruct((_NW, _L), jnp.float32),
        mesh=plsc.VectorSubcoreMesh(core_axis_name="c", subcore_axis_name="s"),
        compiler_params=pltpu.CompilerParams(needs_layout_passes=False),
        scratch_types=[
            pltpu.VMEM((2 * _SB, 128), jnp.float32),
            pltpu.VMEM((2 * _SB, 128), jnp.float32),
            pltpu.VMEM((2 * _PB, 128), jnp.int32),
            pltpu.VMEM((_L,), jnp.float32),
            pltpu.SemaphoreType.DMA,
            pltpu.SemaphoreType.DMA,
            pltpu.SemaphoreType.DMA,
        ],
    )(pv, tv, av)
    return jnp.sum(partials)
